# baseline (speedup 1.0000x reference)
import jax
import jax.numpy as jnp
from jax.experimental import pallas as pl
from jax.experimental.pallas import tpu as pltpu

THR = 1e-20


def _pick_tile(E):
    for t in (400, 200, 8):
        if E % t == 0:
            return t
    return E


def _attn_body(qq_ref, emb_ref, WihT_ref, WhhT_ref, bih_ref, bhh_ref,
               WlinT_ref, blin_ref, out_ref):
    B = qq_ref.shape[1]
    NQ1, EM = emb_ref.shape
    K, _, _, H4 = WihT_ref.shape
    H = H4 // 4

    qv = qq_ref[0, :]
    cols = jax.lax.broadcasted_iota(jnp.int32, (B, NQ1), 1)
    oh = jnp.where(cols == qv[:, None], 1.0, 0.0).astype(jnp.float32)
    q01 = jnp.dot(oh, emb_ref[...], preferred_element_type=jnp.float32)
    q2 = jnp.broadcast_to(emb_ref[NQ1 - 1:NQ1, :], (B, EM))
    xs_f = (q01, q01, q2)
    xs_b = (q2, q01, q01)

    for k in range(K):
        hs = [[None] * 3, [None] * 3]
        for d, xs in ((0, xs_f), (1, xs_b)):
            h = jnp.zeros((B, H), jnp.float32)
            c = jnp.zeros((B, H), jnp.float32)
            for t in range(3):
                g = (jnp.dot(xs[t], WihT_ref[k, d], preferred_element_type=jnp.float32)
                     + jnp.dot(h, WhhT_ref[k, d], preferred_element_type=jnp.float32)
                     + bih_ref[k, d] + bhh_ref[k, d])
                ig = jax.nn.sigmoid(g[:, 0:H])
                fg = jax.nn.sigmoid(g[:, H:2 * H])
                gg = jnp.tanh(g[:, 2 * H:3 * H])
                og = jax.nn.sigmoid(g[:, 3 * H:4 * H])
                c = fg * c + ig * gg
                h = og * jnp.tanh(c)
                hs[d][t] = h
        for t in range(2):
            out_t = jnp.concatenate([hs[0][t], hs[1][2 - t]], axis=1)
            logits = jnp.dot(out_t, WlinT_ref[...],
                             preferred_element_type=jnp.float32) + blin_ref[0, :]
            m = jnp.max(logits, axis=1, keepdims=True)
            e = jnp.exp(logits - m)
            a = e / jnp.sum(e, axis=1, keepdims=True)
            out_ref[k, t, :, :] = a


def _attention(qq, emb, Wih, Whh, bih, bhh, Wlin, blin):
    K, _, H4, EM = Wih.shape
    B = qq.shape[0]
    NOP1 = Wlin.shape[0]
    return pl.pallas_call(
        _attn_body,
        out_shape=jax.ShapeDtypeStruct((K, 2, B, NOP1), jnp.float32),
    )(qq.reshape(1, B).astype(jnp.int32), emb,
      Wih.transpose(0, 1, 3, 2), Whh.transpose(0, 1, 3, 2),
      bih.reshape(K, 2, 1, H4), bhh.reshape(K, 2, 1, H4),
      Wlin.T, blin.reshape(1, NOP1))


def _onehot_body(tt_ref, out_ref):
    T, C = out_ref.shape
    B = tt_ref.shape[1]
    i = pl.program_id(0)
    rows = i * T + jax.lax.broadcasted_iota(jnp.int32, (T, B), 0)
    m = jnp.where(rows == tt_ref[0, :][None, :], 1.0, 0.0).astype(jnp.float32)
    out_ref[...] = jnp.concatenate([m] * (C // B), axis=1)


def _onehot_rhs(tt, E, K):
    B = tt.shape[0]
    T = _pick_tile(E)
    return pl.pallas_call(
        _onehot_body,
        grid=(E // T,),
        in_specs=[pl.BlockSpec((1, B), lambda i: (0, 0))],
        out_specs=pl.BlockSpec((T, K * B), lambda i: (i, 0)),
        out_shape=jax.ShapeDtypeStruct((E, K * B), jnp.float32),
    )(tt.reshape(1, B).astype(jnp.int32))


def _prop_body(mdb_ref, rhs_ref, Ar_ref, Askip_ref, normin_ref,
               acc_ref, norms_ref):
    T = acc_ref.shape[0]
    i = pl.program_id(0)
    r = pl.program_id(1)
    R = pl.num_programs(1)
    ninv = 1.0 / jnp.maximum(normin_ref[0, :], THR)
    w = Ar_ref[0, 0, :] * ninv
    part = jnp.dot(mdb_ref[0], rhs_ref[...],
                   preferred_element_type=jnp.float32) * w[None, :]

    @pl.when(r == 0)
    def _():
        wskip = Askip_ref[0, :] * ninv
        acc_ref[...] = part + rhs_ref[pl.ds(i * T, T), :] * wskip[None, :]

    @pl.when(r > 0)
    def _():
        acc_ref[...] += part

    @pl.when(r == R - 1)
    def _():
        colsum = jnp.sum(jnp.abs(acc_ref[...]), axis=0, keepdims=True)

        @pl.when(i == 0)
        def _():
            norms_ref[...] = colsum

        @pl.when(i > 0)
        def _():
            norms_ref[...] += colsum


def _propagate(mdb, rhs, A, norm_in):
    R, E, _ = mdb.shape
    C = rhs.shape[1]
    T = _pick_tile(E)
    Ar = A.T.reshape(R + 1, 1, C)
    return pl.pallas_call(
        _prop_body,
        grid=(E // T, R),
        in_specs=[
            pl.BlockSpec((1, T, E), lambda i, r: (r, i, 0)),
            pl.BlockSpec((E, C), lambda i, r: (0, 0)),
            pl.BlockSpec((1, 1, C), lambda i, r: (r, 0, 0)),
            pl.BlockSpec((1, C), lambda i, r: (0, 0)),
            pl.BlockSpec((1, C), lambda i, r: (0, 0)),
        ],
        out_specs=[
            pl.BlockSpec((T, C), lambda i, r: (i, 0)),
            pl.BlockSpec((1, C), lambda i, r: (0, 0)),
        ],
        out_shape=[
            jax.ShapeDtypeStruct((E, C), jnp.float32),
            jax.ShapeDtypeStruct((1, C), jnp.float32),
        ],
        compiler_params=pltpu.CompilerParams(
            dimension_semantics=("arbitrary", "arbitrary")),
    )(mdb, rhs, Ar[:R], Ar[R], norm_in)


def _epi_body(acc_ref, norms_ref, out_ref):
    B = out_ref.shape[1]
    K = acc_ref.shape[1] // B
    t = acc_ref[...] / jnp.maximum(norms_ref[0, :], THR)[None, :]
    s = t[:, 0:B]
    for k in range(1, K):
        s = s + t[:, k * B:(k + 1) * B]
    out_ref[...] = s


def _epilogue(acc, norms, B):
    E, C = acc.shape
    T = _pick_tile(E)
    return pl.pallas_call(
        _epi_body,
        grid=(E // T,),
        in_specs=[
            pl.BlockSpec((T, C), lambda i: (i, 0)),
            pl.BlockSpec((1, C), lambda i: (0, 0)),
        ],
        out_specs=pl.BlockSpec((T, B), lambda i: (i, 0)),
        out_shape=jax.ShapeDtypeStruct((E, B), jnp.float32),
    )(acc, norms)


def kernel(qq, tt, mdb, emb, Wih, Whh, bih, bhh, Wlin, blin):
    R, E, _ = mdb.shape
    B = qq.shape[0]
    K = Wih.shape[0]
    C = K * B

    attn = _attention(qq, emb, Wih, Whh, bih, bhh, Wlin, blin)
    A0 = attn[:, 0].reshape(C, R + 1)
    A1 = attn[:, 1].reshape(C, R + 1)

    rhs0 = _onehot_rhs(tt, E, K)
    ones = jnp.ones((1, C), jnp.float32)
    acc0, norms0 = _propagate(mdb, rhs0, A0, ones)
    acc1, norms1 = _propagate(mdb, acc0, A1, norms0)
    predT = _epilogue(acc1, norms1, B)
    return predT.T
